# baseline (speedup 1.0000x reference)
import functools

import jax
import jax.numpy as jnp
from jax import lax
from jax.experimental import pallas as pl
from jax.experimental.pallas import tpu as pltpu
from jax.experimental.pallas import tpu_sc as plsc

NC = 2
NS = 16
NW = NC * NS

B_ROWS = 4096 * 26
D = 64
B_PER_W = B_ROWS // NW
CHUNK = 128
N_CHUNKS = B_PER_W // CHUNK
N_PAIRS = N_CHUNKS // 2

_mesh = plsc.VectorSubcoreMesh(core_axis_name="c", subcore_axis_name="s")


@functools.partial(
    pl.kernel,
    mesh=_mesh,
    out_type=jax.ShapeDtypeStruct((B_ROWS, D), jnp.float32),
    scratch_types=[
        pltpu.VMEM((N_CHUNKS, CHUNK), jnp.int32),
        pltpu.VMEM((CHUNK, D), jnp.float32),
        pltpu.VMEM((CHUNK, D), jnp.float32),
        pltpu.SemaphoreType.DMA,
        pltpu.SemaphoreType.DMA,
    ],
    compiler_params=pltpu.CompilerParams(use_tc_tiling_on_sc=False),
)
def _gather_sc(table_hbm, idx_hbm, out_hbm, idx_v, rows0, rows1, sem0, sem1):
    wid = lax.axis_index("s") * NC + lax.axis_index("c")
    base = wid * B_PER_W
    pltpu.sync_copy(idx_hbm.at[wid], idx_v)

    def pair(g, carry):
        j0 = 2 * g
        cp0 = pltpu.async_copy(table_hbm.at[idx_v.at[j0]], rows0, sem0)
        cp1 = pltpu.async_copy(table_hbm.at[idx_v.at[j0 + 1]], rows1, sem1)
        cp0.wait()
        pltpu.sync_copy(rows0, out_hbm.at[pl.ds(base + j0 * CHUNK, CHUNK)])
        cp1.wait()
        pltpu.sync_copy(rows1, out_hbm.at[pl.ds(base + (j0 + 1) * CHUNK, CHUNK)])
        return carry

    lax.fori_loop(0, N_PAIRS, pair, 0)


def kernel(input_tensor, indices):
    idx = indices.reshape(NW, N_CHUNKS, CHUNK).astype(jnp.int32)
    out = _gather_sc(input_tensor, idx)
    return out.reshape(indices.shape[0], indices.shape[1], D)


# speedup vs baseline: 1.0060x; 1.0060x over previous
import functools

import jax
import jax.numpy as jnp
from jax import lax
from jax.experimental import pallas as pl
from jax.experimental.pallas import tpu as pltpu
from jax.experimental.pallas import tpu_sc as plsc

NC = 2
NS = 16
NW = NC * NS

B_ROWS = 4096 * 26
D = 64
B_PER_W = B_ROWS // NW
CHUNK = 128
N_CHUNKS = B_PER_W // CHUNK
N_PAIRS = N_CHUNKS // 2

_mesh = plsc.VectorSubcoreMesh(core_axis_name="c", subcore_axis_name="s")


@functools.partial(
    pl.kernel,
    mesh=_mesh,
    out_type=jax.ShapeDtypeStruct((B_ROWS, D), jnp.float32),
    scratch_types=[
        pltpu.VMEM((N_CHUNKS, CHUNK), jnp.int32),
    ]
    + [pltpu.VMEM((CHUNK, D), jnp.float32) for _ in range(4)]
    + [pltpu.SemaphoreType.DMA for _ in range(8)],
    compiler_params=pltpu.CompilerParams(use_tc_tiling_on_sc=False),
)
def _gather_sc(table_hbm, idx_hbm, out_hbm, idx_v, *scratch):
    NBUF = 4
    W_LAG = 2
    rows = scratch[:NBUF]
    gsem = scratch[NBUF:2 * NBUF]
    wsem = scratch[2 * NBUF:3 * NBUF]

    wid = lax.axis_index("s") * NC + lax.axis_index("c")
    base = wid * B_PER_W
    pltpu.sync_copy(idx_hbm.at[wid], idx_v)

    def start_gather(j, b):
        return pltpu.async_copy(table_hbm.at[idx_v.at[j]], rows[b], gsem[b])

    def start_write(j, b):
        return pltpu.async_copy(
            rows[b], out_hbm.at[pl.ds(base + j * CHUNK, CHUNK)], wsem[b])

    gathers = {}
    writes = {}
    for b in range(NBUF):
        gathers[b] = start_gather(b, b)
    for j in range(N_CHUNKS):
        b = j % NBUF
        gathers[b].wait()
        writes[b] = start_write(j, b)
        k = j - W_LAG
        nk = k + NBUF
        if k >= 0 and nk < N_CHUNKS:
            kb = k % NBUF
            writes[kb].wait()
            gathers[kb] = start_gather(nk, kb)
    for j in range(min(N_CHUNKS - W_LAG, N_CHUNKS - NBUF), N_CHUNKS):
        writes[j % NBUF].wait()


def kernel(input_tensor, indices):
    idx = indices.reshape(NW, N_CHUNKS, CHUNK).astype(jnp.int32)
    out = _gather_sc(input_tensor, idx)
    return out.reshape(indices.shape[0], indices.shape[1], D)


# speedup vs baseline: 1.0572x; 1.0508x over previous
import functools

import jax
import jax.numpy as jnp
from jax import lax
from jax.experimental import pallas as pl
from jax.experimental.pallas import tpu as pltpu
from jax.experimental.pallas import tpu_sc as plsc

NC = 2
NS = 16
NW = NC * NS

B_ROWS = 4096 * 26
D = 64
DP = 128
B_PER_W = B_ROWS // NW
CHUNK = 128
N_CHUNKS = B_PER_W // CHUNK

_mesh = plsc.VectorSubcoreMesh(core_axis_name="c", subcore_axis_name="s")


@functools.partial(
    pl.kernel,
    mesh=_mesh,
    out_type=jax.ShapeDtypeStruct((B_ROWS, DP), jnp.float32),
    scratch_types=[
        pltpu.VMEM((N_CHUNKS, CHUNK), jnp.int32),
    ]
    + [pltpu.VMEM((CHUNK, DP), jnp.float32) for _ in range(4)]
    + [pltpu.SemaphoreType.DMA for _ in range(8)],
)
def _gather_sc(table_hbm, idx_hbm, out_hbm, idx_v, *scratch):
    NBUF = 4
    W_LAG = 2
    rows = scratch[:NBUF]
    gsem = scratch[NBUF:2 * NBUF]
    wsem = scratch[2 * NBUF:3 * NBUF]

    wid = lax.axis_index("s") * NC + lax.axis_index("c")
    base = wid * B_PER_W
    pltpu.sync_copy(idx_hbm.at[wid], idx_v)

    def start_gather(j, b):
        return pltpu.async_copy(table_hbm.at[idx_v.at[j]], rows[b], gsem[b])

    def start_write(j, b):
        return pltpu.async_copy(
            rows[b],
            out_hbm.at[pl.ds(base + j * CHUNK, CHUNK)],
            wsem[b],
        )

    gathers = {}
    writes = {}
    for b in range(NBUF):
        gathers[b] = start_gather(b, b)
    for j in range(N_CHUNKS):
        b = j % NBUF
        gathers[b].wait()
        writes[b] = start_write(j, b)
        k = j - W_LAG
        nk = k + NBUF
        if k >= 0 and nk < N_CHUNKS:
            kb = k % NBUF
            writes[kb].wait()
            gathers[kb] = start_gather(nk, kb)
    for j in range(min(N_CHUNKS - W_LAG, N_CHUNKS - NBUF), N_CHUNKS):
        writes[j % NBUF].wait()


def kernel(input_tensor, indices):
    table_padded = jnp.pad(input_tensor, ((0, 0), (0, DP - D)))
    idx = indices.reshape(NW, N_CHUNKS, CHUNK).astype(jnp.int32)
    out = _gather_sc(table_padded, idx)
    return out[:, :D].reshape(indices.shape[0], indices.shape[1], D)
